# baseline (speedup 1.0000x reference)
import functools

import jax
import jax.numpy as jnp
from jax import lax
from jax.experimental import pallas as pl
from jax.experimental.pallas import tpu as pltpu
from jax.experimental.pallas import tpu_sc as plsc

_NC, _NS, _L = 2, 16, 16
_NW = _NC * _NS
_B = 16384
_K = 64
_BPW = _B // _NW
_CHUNK = 128
_NCHUNK = _BPW // _CHUNK


def _mirt_body(aidx_hbm, tidx_hbm, theta_hbm, a_hbm, d_hbm, out_hbm,
               aidx_v, tidx_v, th_v, av_v, dv_v, out_v, sem):
    wid = lax.axis_index("s") * _NC + lax.axis_index("c")
    base = wid * _BPW
    crow = wid * _NCHUNK

    pltpu.sync_copy(aidx_hbm.at[pl.ds(crow, _NCHUNK)], aidx_v)
    pltpu.sync_copy(tidx_hbm.at[pl.ds(crow, _NCHUNK)], tidx_v)

    cps = []
    for c in range(_NCHUNK):
        lo = c * _CHUNK
        cps.append(pltpu.async_copy(
            theta_hbm.at[aidx_v.at[c]], th_v.at[pl.ds(lo, _CHUNK)], sem))
        cps.append(pltpu.async_copy(
            a_hbm.at[tidx_v.at[c]], av_v.at[pl.ds(lo, _CHUNK)], sem))
        cps.append(pltpu.async_copy(
            d_hbm.at[tidx_v.at[c]], dv_v.at[pl.ds(lo, _CHUNK)], sem))
    for cp in cps:
        cp.wait()

    def block(bi, carry):
        lo = bi * _L
        rows = lo + lax.iota(jnp.int32, _L)
        acc = dv_v[pl.ds(lo, _L)]
        for kk in range(_K):
            cols = jnp.full((_L,), kk, jnp.int32)
            thg = plsc.load_gather(th_v, [rows, cols])
            ag = plsc.load_gather(av_v, [rows, cols])
            acc = acc + thg * ag
        out_v[pl.ds(lo, _L)] = acc
        return carry

    lax.fori_loop(0, _BPW // _L, block, 0)
    pltpu.sync_copy(out_v, out_hbm.at[pl.ds(base, _BPW)])


@jax.jit
def kernel(agent_idx, task_idx, theta, a, d):
    aidx2 = agent_idx.reshape(_NW * _NCHUNK, _CHUNK).astype(jnp.int32)
    tidx2 = task_idx.reshape(_NW * _NCHUNK, _CHUNK).astype(jnp.int32)
    d1 = d.reshape(-1)
    mesh = plsc.VectorSubcoreMesh(core_axis_name="c", subcore_axis_name="s")
    f = pl.kernel(
        _mirt_body,
        out_type=jax.ShapeDtypeStruct((_B,), jnp.float32),
        mesh=mesh,
        compiler_params=pltpu.CompilerParams(
            needs_layout_passes=False, use_tc_tiling_on_sc=False),
        scratch_types=[
            pltpu.VMEM((_NCHUNK, _CHUNK), jnp.int32),
            pltpu.VMEM((_NCHUNK, _CHUNK), jnp.int32),
            pltpu.VMEM((_BPW, _K), jnp.float32),
            pltpu.VMEM((_BPW, _K), jnp.float32),
            pltpu.VMEM((_BPW,), jnp.float32),
            pltpu.VMEM((_BPW,), jnp.float32),
            pltpu.SemaphoreType.DMA,
        ],
    )
    return f(aidx2, tidx2, theta, a, d1)
